# baseline (speedup 1.0000x reference)
import functools

import jax
import jax.numpy as jnp
from jax import lax
from jax.experimental import pallas as pl
from jax.experimental.pallas import tpu as pltpu
from jax.experimental.pallas import tpu_sc as plsc

_NUM_CORES = 2
_NUM_SUBCORES = 16
_NUM_WORKERS = _NUM_CORES * _NUM_SUBCORES
_CHUNK_ROWS = 8
_CHUNK_COLS = 2048
_NBUF = 6
_PRIME = 3


def kernel(inputs):
    B, R, C = inputs.shape
    workers_per_row = _NUM_WORKERS // B
    rpw = R // workers_per_row
    nrow = rpw // _CHUNK_ROWS
    ncol = C // _CHUNK_COLS
    nchunks = nrow * ncol
    assert rpw % _CHUNK_ROWS == 0 and C % _CHUNK_COLS == 0

    perm = jax.random.permutation(jax.random.key(42), B)
    wid = (
        jnp.arange(_NUM_SUBCORES, dtype=jnp.int32)[None, :] * _NUM_CORES
        + jnp.arange(_NUM_CORES, dtype=jnp.int32)[:, None]
    )
    src_batch = perm.astype(jnp.int32)[wid // workers_per_row]
    src_batch = jnp.broadcast_to(
        src_batch[:, :, None], (_NUM_CORES, _NUM_SUBCORES, 16)
    ).astype(jnp.int32)

    mesh = plsc.VectorSubcoreMesh(core_axis_name="c", subcore_axis_name="s")

    @functools.partial(
        pl.kernel,
        out_type=jax.ShapeDtypeStruct((B, R, C), jnp.float32),
        mesh=mesh,
        scratch_types=[
            pltpu.VMEM((16,), jnp.int32),
            *[
                pltpu.VMEM((_CHUNK_ROWS, _CHUNK_COLS), jnp.float32)
                for _ in range(_NBUF)
            ],
            *[pltpu.SemaphoreType.DMA for _ in range(2 * _NBUF)],
        ],
    )
    def run(in_hbm, src_hbm, out_hbm, idx_v, *bufs_and_sems):
        bufs = bufs_and_sems[:_NBUF]
        lsems = bufs_and_sems[_NBUF : 2 * _NBUF]
        ssems = bufs_and_sems[2 * _NBUF :]
        cid = lax.axis_index("c")
        sid = lax.axis_index("s")
        w = sid * _NUM_CORES + cid
        pltpu.sync_copy(src_hbm.at[cid, sid], idx_v)
        src_b = idx_v[...][0]
        dst_b = w // workers_per_row
        r0 = (w % workers_per_row) * rpw

        def chunk_at(ref, batch, k):
            rc, ch = divmod(k, ncol)
            return ref.at[
                batch,
                pl.ds(pl.multiple_of(r0 + rc * _CHUNK_ROWS, 8), _CHUNK_ROWS),
                pl.ds(ch * _CHUNK_COLS, _CHUNK_COLS),
            ]

        loads = [
            pltpu.async_copy(chunk_at(in_hbm, src_b, k), bufs[k % _NBUF], lsems[k % _NBUF])
            for k in range(min(_PRIME, nchunks))
        ]
        stores = []
        for k in range(nchunks):
            b = k % _NBUF
            loads[k].wait()
            stores.append(pltpu.async_copy(bufs[b], chunk_at(out_hbm, dst_b, k), ssems[b]))
            nk = k + _PRIME
            if nk < nchunks:
                nb = nk % _NBUF
                if nk - _NBUF >= 0:
                    stores[nk - _NBUF].wait()
                loads.append(
                    pltpu.async_copy(chunk_at(in_hbm, src_b, nk), bufs[nb], lsems[nb])
                )
        for k in range(max(0, nchunks - _NBUF), nchunks):
            stores[k].wait()

    return run(inputs, src_batch)


# speedup vs baseline: 1.0071x; 1.0071x over previous
import functools

import jax
import jax.numpy as jnp
from jax import lax
from jax.experimental import pallas as pl
from jax.experimental.pallas import tpu as pltpu
from jax.experimental.pallas import tpu_sc as plsc

_NUM_CORES = 2
_NUM_SUBCORES = 16
_NUM_WORKERS = _NUM_CORES * _NUM_SUBCORES
_CHUNK_ROWS = 8
_NBUF = 3


def kernel(inputs):
    B, R, C = inputs.shape
    workers_per_row = _NUM_WORKERS // B
    rpw = R // workers_per_row
    nchunks = rpw // _CHUNK_ROWS
    assert rpw % _CHUNK_ROWS == 0

    perm = jax.random.permutation(jax.random.key(42), B)
    inv_perm = jnp.argsort(perm)
    wid = (
        jnp.arange(_NUM_SUBCORES, dtype=jnp.int32)[None, :] * _NUM_CORES
        + jnp.arange(_NUM_CORES, dtype=jnp.int32)[:, None]
    )
    dst_batch = inv_perm.astype(jnp.int32)[wid // workers_per_row]
    dst_batch = jnp.broadcast_to(
        dst_batch[:, :, None], (_NUM_CORES, _NUM_SUBCORES, 16)
    ).astype(jnp.int32)

    mesh = plsc.VectorSubcoreMesh(core_axis_name="c", subcore_axis_name="s")

    @functools.partial(
        pl.kernel,
        out_type=jax.ShapeDtypeStruct((B, R, C), jnp.float32),
        mesh=mesh,
        scratch_types=[
            pltpu.VMEM((16,), jnp.int32),
            *[pltpu.VMEM((_CHUNK_ROWS, C), jnp.float32) for _ in range(_NBUF)],
            pltpu.SemaphoreType.DMA,
            *[pltpu.SemaphoreType.DMA for _ in range(2 * _NBUF)],
        ],
    )
    def run(in_hbm, dst_hbm, out_hbm, idx_v, *bufs_and_sems):
        bufs = bufs_and_sems[:_NBUF]
        isem = bufs_and_sems[_NBUF]
        lsems = bufs_and_sems[_NBUF + 1 : 2 * _NBUF + 1]
        ssems = bufs_and_sems[2 * _NBUF + 1 :]
        cid = lax.axis_index("c")
        sid = lax.axis_index("s")
        w = sid * _NUM_CORES + cid
        src_b = w // workers_per_row
        r0 = (w % workers_per_row) * rpw

        idx_cp = pltpu.async_copy(dst_hbm.at[cid, sid], idx_v, isem)

        def src_at(k):
            return in_hbm.at[
                src_b, pl.ds(pl.multiple_of(r0 + k * _CHUNK_ROWS, 8), _CHUNK_ROWS), :
            ]

        loads = [
            pltpu.async_copy(src_at(k), bufs[k], lsems[k])
            for k in range(min(_NBUF, nchunks))
        ]

        idx_cp.wait()
        dst_b = idx_v[...][0]

        def dst_at(k):
            return out_hbm.at[
                dst_b, pl.ds(pl.multiple_of(r0 + k * _CHUNK_ROWS, 8), _CHUNK_ROWS), :
            ]

        stores = []
        for k in range(nchunks):
            b = k % _NBUF
            loads[k].wait()
            stores.append(pltpu.async_copy(bufs[b], dst_at(k), ssems[b]))
            nk = k + _NBUF
            if nk < nchunks:
                stores[k].wait()
                loads.append(pltpu.async_copy(src_at(nk), bufs[b], lsems[b]))
        for k in range(max(0, nchunks - _NBUF), nchunks):
            stores[k].wait()

    return run(inputs, dst_batch)


# speedup vs baseline: 1.0699x; 1.0623x over previous
import functools

import jax
import jax.numpy as jnp
from jax import lax
from jax.experimental import pallas as pl
from jax.experimental.pallas import tpu as pltpu
from jax.experimental.pallas import tpu_sc as plsc

_NUM_CORES = 2
_NUM_SUBCORES = 16
_NUM_WORKERS = _NUM_CORES * _NUM_SUBCORES
_CHUNK_ROWS = 8
_NBUF = 2


def kernel(inputs):
    B, R, C = inputs.shape
    workers_per_row = _NUM_WORKERS // B
    rpw = R // workers_per_row
    nchunks = rpw // _CHUNK_ROWS
    assert rpw % _CHUNK_ROWS == 0

    perm = jax.random.permutation(jax.random.key(42), B)
    inv_perm = jnp.argsort(perm)
    wid = (
        jnp.arange(_NUM_SUBCORES, dtype=jnp.int32)[None, :] * _NUM_CORES
        + jnp.arange(_NUM_CORES, dtype=jnp.int32)[:, None]
    )
    dst_batch = inv_perm.astype(jnp.int32)[wid // workers_per_row]
    dst_batch = jnp.broadcast_to(
        dst_batch[:, :, None], (_NUM_CORES, _NUM_SUBCORES, 16)
    ).astype(jnp.int32)

    mesh = plsc.VectorSubcoreMesh(core_axis_name="c", subcore_axis_name="s")

    @functools.partial(
        pl.kernel,
        out_type=jax.ShapeDtypeStruct((B, R, C), jnp.float32),
        mesh=mesh,
        scratch_types=[
            pltpu.VMEM((16,), jnp.int32),
            pltpu.VMEM_SHARED(
                (_NUM_SUBCORES, _NBUF, _CHUNK_ROWS, 4096), jnp.float32
            ),
            pltpu.SemaphoreType.DMA,
            *[pltpu.SemaphoreType.DMA for _ in range(2 * _NBUF)],
        ],
    )
    def run(in_hbm, dst_hbm, out_hbm, idx_v, shared, *bufs_and_sems):
        isem = bufs_and_sems[0]
        lsems = bufs_and_sems[1 : _NBUF + 1]
        ssems = bufs_and_sems[_NBUF + 1 :]
        cid = lax.axis_index("c")
        sid = lax.axis_index("s")
        bufs = [shared.at[sid, b] for b in range(_NBUF)]
        w = sid * _NUM_CORES + cid
        src_b = w // workers_per_row
        r0 = (w % workers_per_row) * rpw

        idx_cp = pltpu.async_copy(dst_hbm.at[cid, sid], idx_v, isem)

        def src_at(k):
            return in_hbm.at[
                src_b, pl.ds(pl.multiple_of(r0 + k * _CHUNK_ROWS, 8), _CHUNK_ROWS), :
            ]

        loads = [
            pltpu.async_copy(src_at(k), bufs[k], lsems[k])
            for k in range(min(_NBUF, nchunks))
        ]

        idx_cp.wait()
        dst_b = idx_v[...][0]

        def dst_at(k):
            return out_hbm.at[
                dst_b, pl.ds(pl.multiple_of(r0 + k * _CHUNK_ROWS, 8), _CHUNK_ROWS), :
            ]

        stores = []
        for k in range(nchunks):
            b = k % _NBUF
            loads[k].wait()
            stores.append(pltpu.async_copy(bufs[b], dst_at(k), ssems[b]))
            nk = k + _NBUF
            if nk < nchunks:
                stores[k].wait()
                loads.append(pltpu.async_copy(src_at(nk), bufs[b], lsems[b]))
        for k in range(max(0, nchunks - _NBUF), nchunks):
            stores[k].wait()

    return run(inputs, dst_batch)


# speedup vs baseline: 1.0742x; 1.0040x over previous
import functools

import jax
import jax.numpy as jnp
from jax import lax
from jax.experimental import pallas as pl
from jax.experimental.pallas import tpu as pltpu
from jax.experimental.pallas import tpu_sc as plsc

_NUM_CORES = 2
_NUM_SUBCORES = 16
_NUM_WORKERS = _NUM_CORES * _NUM_SUBCORES
_CHUNK_ROWS = 8
_NBUF = 3


def kernel(inputs):
    B, R, C = inputs.shape
    workers_per_row = _NUM_WORKERS // B
    rpw = R // workers_per_row
    nchunks = rpw // _CHUNK_ROWS
    assert rpw % _CHUNK_ROWS == 0

    perm = jax.random.permutation(jax.random.key(42), B)
    inv_perm = jnp.argsort(perm)
    wid = (
        jnp.arange(_NUM_SUBCORES, dtype=jnp.int32)[None, :] * _NUM_CORES
        + jnp.arange(_NUM_CORES, dtype=jnp.int32)[:, None]
    )
    dst_batch = inv_perm.astype(jnp.int32)[wid // workers_per_row]
    dst_batch = jnp.broadcast_to(
        dst_batch[:, :, None], (_NUM_CORES, _NUM_SUBCORES, 16)
    ).astype(jnp.int32)

    mesh = plsc.VectorSubcoreMesh(core_axis_name="c", subcore_axis_name="s")

    @functools.partial(
        pl.kernel,
        out_type=jax.ShapeDtypeStruct((B, R, C), jnp.float32),
        mesh=mesh,
        scratch_types=[
            pltpu.VMEM((16,), jnp.int32),
            pltpu.VMEM_SHARED(
                (_NUM_SUBCORES, _NBUF, _CHUNK_ROWS, 4096), jnp.float32
            ),
            pltpu.SemaphoreType.DMA,
            *[pltpu.SemaphoreType.DMA for _ in range(2 * _NBUF)],
        ],
    )
    def run(in_hbm, dst_hbm, out_hbm, idx_v, shared, *bufs_and_sems):
        isem = bufs_and_sems[0]
        lsems = bufs_and_sems[1 : _NBUF + 1]
        ssems = bufs_and_sems[_NBUF + 1 :]
        cid = lax.axis_index("c")
        sid = lax.axis_index("s")
        bufs = [shared.at[sid, b] for b in range(_NBUF)]
        w = sid * _NUM_CORES + cid
        src_b = w // workers_per_row
        r0 = (w % workers_per_row) * rpw

        idx_cp = pltpu.async_copy(dst_hbm.at[cid, sid], idx_v, isem)

        def src_at(k):
            return in_hbm.at[
                src_b, pl.ds(pl.multiple_of(r0 + k * _CHUNK_ROWS, 8), _CHUNK_ROWS), :
            ]

        loads = [
            pltpu.async_copy(src_at(k), bufs[k], lsems[k])
            for k in range(min(_NBUF, nchunks))
        ]

        idx_cp.wait()
        dst_b = idx_v[...][0]

        def dst_at(k):
            return out_hbm.at[
                dst_b, pl.ds(pl.multiple_of(r0 + k * _CHUNK_ROWS, 8), _CHUNK_ROWS), :
            ]

        stores = []
        for k in range(nchunks):
            b = k % _NBUF
            loads[k].wait()
            stores.append(pltpu.async_copy(bufs[b], dst_at(k), ssems[b]))
            nk = k + _NBUF
            if nk < nchunks:
                stores[k].wait()
                loads.append(pltpu.async_copy(src_at(nk), bufs[b], lsems[b]))
        for k in range(max(0, nchunks - _NBUF), nchunks):
            stores[k].wait()

    return run(inputs, dst_batch)


# speedup vs baseline: 1.0766x; 1.0023x over previous
import functools

import jax
import jax.numpy as jnp
from jax import lax
from jax.experimental import pallas as pl
from jax.experimental.pallas import tpu as pltpu
from jax.experimental.pallas import tpu_sc as plsc

_NUM_CORES = 2
_NUM_SUBCORES = 16
_NUM_WORKERS = _NUM_CORES * _NUM_SUBCORES
_CHUNK_ROWS = 8
_NBUF = 3


def kernel(inputs):
    B, R, C = inputs.shape
    workers_per_row = _NUM_WORKERS // B
    rpw = R // workers_per_row
    nchunks = rpw // _CHUNK_ROWS
    assert rpw % _CHUNK_ROWS == 0

    perm = jax.random.permutation(jax.random.key(42), B)
    inv_perm = jnp.argsort(perm)
    wid = (
        jnp.arange(_NUM_SUBCORES, dtype=jnp.int32)[None, :] * _NUM_CORES
        + jnp.arange(_NUM_CORES, dtype=jnp.int32)[:, None]
    )
    dst_batch = inv_perm.astype(jnp.int32)[wid // workers_per_row]
    dst_batch = jnp.broadcast_to(
        dst_batch[:, :, None], (_NUM_CORES, _NUM_SUBCORES, 16)
    ).astype(jnp.int32)

    mesh = plsc.VectorSubcoreMesh(core_axis_name="c", subcore_axis_name="s")

    @functools.partial(
        pl.kernel,
        out_type=jax.ShapeDtypeStruct((B, R, C), jnp.float32),
        mesh=mesh,
        scratch_types=[
            pltpu.VMEM((16,), jnp.int32),
            pltpu.VMEM_SHARED(
                (_NUM_SUBCORES, _NBUF, _CHUNK_ROWS, 4096), jnp.float32
            ),
            pltpu.SemaphoreType.DMA,
            *[pltpu.SemaphoreType.DMA for _ in range(2 * _NBUF)],
        ],
    )
    def run(in_hbm, dst_hbm, out_hbm, idx_v, shared, *sems):
        isem = sems[0]
        lsems = sems[1 : _NBUF + 1]
        ssems = sems[_NBUF + 1 :]
        cid = lax.axis_index("c")
        sid = lax.axis_index("s")
        bufs = [shared.at[sid, b] for b in range(_NBUF)]
        w = sid * _NUM_CORES + cid
        src_b = w // workers_per_row
        r0 = (w % workers_per_row) * rpw

        idx_cp = pltpu.async_copy(dst_hbm.at[cid, sid], idx_v, isem)

        def src_at(k):
            return in_hbm.at[
                src_b, pl.ds(pl.multiple_of(r0 + k * _CHUNK_ROWS, 8), _CHUNK_ROWS), :
            ]

        loads = [
            pltpu.async_copy(src_at(k), bufs[k], lsems[k])
            for k in range(min(_NBUF, nchunks))
        ]

        idx_cp.wait()
        dst_b = idx_v[...][0]

        def dst_at(k):
            return out_hbm.at[
                dst_b, pl.ds(pl.multiple_of(r0 + k * _CHUNK_ROWS, 8), _CHUNK_ROWS), :
            ]

        stores = []
        for k in range(nchunks):
            b = k % _NBUF
            loads[k].wait()
            stores.append(pltpu.async_copy(bufs[b], dst_at(k), ssems[b]))
            nk = k + _NBUF
            if nk < nchunks:
                stores[k].wait()
                loads.append(pltpu.async_copy(src_at(nk), bufs[b], lsems[b]))
        for k in range(max(0, nchunks - _NBUF), nchunks):
            stores[k].wait()

    return run(inputs, dst_batch)
